# Initial kernel scaffold; baseline (speedup 1.0000x reference)
import jax
import jax.numpy as jnp
from jax.experimental import pallas as pl


def kernel(x, edge_index, W1, b1, W2, b2):
    raise NotImplementedError("write your pallas kernel here")


# speedup vs baseline: 186.4380x; 186.4380x over previous
import functools
import jax
import jax.numpy as jnp
from jax import lax
from jax.experimental import pallas as pl
from jax.experimental.pallas import tpu as pltpu
from jax.experimental.pallas import tpu_sc as plsc

N_NODES = 100000
N_PAD = 102400
N_TILES = 32
E = 3_200_000
EPT = E // N_TILES
CH = 10000
ROWS = N_PAD // 128
GBLK = 200

_mesh = plsc.VectorSubcoreMesh(core_axis_name="c", subcore_axis_name="s")
_cp = pltpu.CompilerParams(needs_layout_passes=False)


@functools.partial(
    pl.kernel, mesh=_mesh, compiler_params=_cp,
    out_type=jax.ShapeDtypeStruct((N_TILES, N_PAD), jnp.float32),
    scratch_types=[
        pltpu.VMEM((N_PAD,), jnp.float32),
        pltpu.VMEM((CH,), jnp.int32),
    ],
)
def _hist(dst_hbm, out_hbm, acc_v, idx_v):
    wid = lax.axis_index("s") * 2 + lax.axis_index("c")
    zero = jnp.zeros((16,), jnp.float32)

    def zbody(i, carry):
        acc_v[pl.ds(i * 16, 16)] = zero
        return carry
    lax.fori_loop(0, N_PAD // 16, zbody, 0)

    ones = jnp.ones((16,), jnp.float32)
    base = wid * EPT

    def chunk(ci, carry):
        pltpu.sync_copy(dst_hbm.at[pl.ds(base + ci * CH, CH)], idx_v)

        def step(i, c2):
            d = idx_v[pl.ds(i * 16, 16)]
            plsc.addupdate_scatter(acc_v, [d], ones)
            return c2
        lax.fori_loop(0, CH // 16, step, 0)
        return carry
    lax.fori_loop(0, EPT // CH, chunk, 0)
    pltpu.sync_copy(acc_v, out_hbm.at[wid])


@functools.partial(
    pl.kernel, mesh=_mesh, compiler_params=_cp,
    out_type=jax.ShapeDtypeStruct((E,), jnp.float32),
    scratch_types=[
        pltpu.VMEM((N_PAD,), jnp.float32),
        pltpu.VMEM((CH,), jnp.int32),
        pltpu.VMEM((CH,), jnp.float32),
    ],
)
def _gather(g_hbm, src_hbm, msg_hbm, g_v, idx_v, val_v):
    wid = lax.axis_index("s") * 2 + lax.axis_index("c")
    pltpu.sync_copy(g_hbm, g_v)
    base = wid * EPT

    def chunk(ci, carry):
        off = base + ci * CH
        pltpu.sync_copy(src_hbm.at[pl.ds(off, CH)], idx_v)

        def step(i, c2):
            s = idx_v[pl.ds(i * 16, 16)]
            val_v[pl.ds(i * 16, 16)] = plsc.load_gather(g_v, [s])
            return c2
        lax.fori_loop(0, CH // 16, step, 0)
        pltpu.sync_copy(val_v, msg_hbm.at[pl.ds(off, CH)])
        return carry
    lax.fori_loop(0, EPT // CH, chunk, 0)


@functools.partial(
    pl.kernel, mesh=_mesh, compiler_params=_cp,
    out_type=jax.ShapeDtypeStruct((N_TILES, N_PAD), jnp.float32),
    scratch_types=[
        pltpu.VMEM((N_PAD,), jnp.float32),
        pltpu.VMEM((CH,), jnp.int32),
        pltpu.VMEM((CH,), jnp.float32),
    ],
)
def _scatter(dst_hbm, msg_hbm, out_hbm, acc_v, idx_v, val_v):
    wid = lax.axis_index("s") * 2 + lax.axis_index("c")
    zero = jnp.zeros((16,), jnp.float32)

    def zbody(i, carry):
        acc_v[pl.ds(i * 16, 16)] = zero
        return carry
    lax.fori_loop(0, N_PAD // 16, zbody, 0)

    base = wid * EPT

    def chunk(ci, carry):
        off = base + ci * CH
        pltpu.sync_copy(dst_hbm.at[pl.ds(off, CH)], idx_v)
        pltpu.sync_copy(msg_hbm.at[pl.ds(off, CH)], val_v)

        def step(i, c2):
            d = idx_v[pl.ds(i * 16, 16)]
            m = val_v[pl.ds(i * 16, 16)]
            plsc.addupdate_scatter(acc_v, [d], m)
            return c2
        lax.fori_loop(0, CH // 16, step, 0)
        return carry
    lax.fori_loop(0, EPT // CH, chunk, 0)
    pltpu.sync_copy(acc_v, out_hbm.at[wid])


def _glue1_body(w_ref, degp_ref, x0_ref, x1_ref, dinv_ref, g1_ref):
    deg = jnp.sum(degp_ref[...], axis=0) + 1.0
    dinv = jax.lax.rsqrt(deg)
    z0 = x0_ref[...] * w_ref[0] + x1_ref[...] * w_ref[1]
    dinv_ref[...] = dinv
    g1_ref[...] = dinv * z0


def _glue2_body(c_ref, accp_ref, g1_ref, dinv_ref, g2_ref):
    s = jnp.sum(accp_ref[...], axis=0) + g1_ref[...]
    dinv = dinv_ref[...]
    z1 = dinv * s + c_ref[0]
    g2_ref[...] = dinv * z1


def _glue3_body(b_ref, accp_ref, g2_ref, dinv_ref, out_ref):
    s = jnp.sum(accp_ref[...], axis=0) + g2_ref[...]
    out_ref[...] = dinv_ref[...] * s + b_ref[0]


def _part_spec():
    return pl.BlockSpec((N_TILES, GBLK, 128), lambda i: (0, i, 0))


def _row_spec():
    return pl.BlockSpec((GBLK, 128), lambda i: (i, 0))


def _smem_spec():
    return pl.BlockSpec(memory_space=pltpu.SMEM)


_f2d = jax.ShapeDtypeStruct((ROWS, 128), jnp.float32)
_grid = ROWS // GBLK

_glue1 = pl.pallas_call(
    _glue1_body, grid=(_grid,),
    in_specs=[_smem_spec(), _part_spec(), _row_spec(), _row_spec()],
    out_specs=[_row_spec(), _row_spec()],
    out_shape=[_f2d, _f2d],
)

_glue2 = pl.pallas_call(
    _glue2_body, grid=(_grid,),
    in_specs=[_smem_spec(), _part_spec(), _row_spec(), _row_spec()],
    out_specs=[_row_spec()],
    out_shape=[_f2d],
)

_glue3 = pl.pallas_call(
    _glue3_body, grid=(_grid,),
    in_specs=[_smem_spec(), _part_spec(), _row_spec(), _row_spec()],
    out_specs=[_row_spec()],
    out_shape=[_f2d],
)


def kernel(x, edge_index, W1, b1, W2, b2):
    src = edge_index[0].astype(jnp.int32)
    dst = edge_index[1].astype(jnp.int32)
    w = (W1 @ W2)[:, 0]
    c = jnp.dot(b1, W2[:, 0])

    xp = jnp.pad(x, ((0, N_PAD - N_NODES), (0, 0)))
    x0 = xp[:, 0].reshape(ROWS, 128)
    x1 = xp[:, 1].reshape(ROWS, 128)

    degp = _hist(dst).reshape(N_TILES, ROWS, 128)
    dinv, g1 = _glue1(w, degp, x0, x1)

    msg1 = _gather(g1.reshape(N_PAD), src)
    accp1 = _scatter(dst, msg1).reshape(N_TILES, ROWS, 128)
    (g2,) = _glue2(jnp.reshape(c, (1,)), accp1, g1, dinv)

    msg2 = _gather(g2.reshape(N_PAD), src)
    accp2 = _scatter(dst, msg2).reshape(N_TILES, ROWS, 128)
    (out,) = _glue3(b2, accp2, g2, dinv)

    return out.reshape(N_PAD)[:N_NODES]


# speedup vs baseline: 227.1218x; 1.2182x over previous
import functools
import jax
import jax.numpy as jnp
from jax import lax
from jax.experimental import pallas as pl
from jax.experimental.pallas import tpu as pltpu
from jax.experimental.pallas import tpu_sc as plsc

N_NODES = 100000
N_PAD = 102400
N_TILES = 32
E = 3_200_000
EPT = E // N_TILES
CH = 10000
UN = 25
BK = N_PAD // 4

_mesh = plsc.VectorSubcoreMesh(core_axis_name="c", subcore_axis_name="s")
_cp = pltpu.CompilerParams(needs_layout_passes=False)


def _zero_acc(acc_v):
    zero = jnp.zeros((16,), jnp.float32)

    def zbody(i, carry):
        for u in range(16):
            acc_v[pl.ds(i * 256 + u * 16, 16)] = zero
        return carry
    lax.fori_loop(0, N_PAD // 256, zbody, 0)


@functools.partial(
    pl.kernel, mesh=_mesh, compiler_params=_cp,
    out_type=jax.ShapeDtypeStruct((N_TILES, N_PAD), jnp.float32),
    scratch_types=[
        pltpu.VMEM((N_PAD,), jnp.float32),
        pltpu.VMEM((CH,), jnp.int32),
    ],
)
def _hist(dst_hbm, out_hbm, acc_v, idx_v):
    wid = lax.axis_index("s") * 2 + lax.axis_index("c")
    _zero_acc(acc_v)
    ones = jnp.ones((16,), jnp.float32)
    base = wid * EPT

    def chunk(ci, carry):
        pltpu.sync_copy(dst_hbm.at[pl.ds(base + ci * CH, CH)], idx_v)

        def step(i, c2):
            for u in range(UN):
                d = idx_v[pl.ds(i * (16 * UN) + u * 16, 16)]
                plsc.addupdate_scatter(acc_v, [d], ones)
            return c2
        lax.fori_loop(0, CH // (16 * UN), step, 0)
        return carry
    lax.fori_loop(0, EPT // CH, chunk, 0)
    pltpu.sync_copy(acc_v, out_hbm.at[wid])


@functools.partial(
    pl.kernel, mesh=_mesh, compiler_params=_cp,
    out_type=jax.ShapeDtypeStruct((E,), jnp.float32),
    scratch_types=[
        pltpu.VMEM((N_PAD,), jnp.float32),
        pltpu.VMEM((CH,), jnp.int32),
        pltpu.VMEM((CH,), jnp.float32),
    ],
)
def _gather(g_hbm, src_hbm, msg_hbm, g_v, idx_v, val_v):
    wid = lax.axis_index("s") * 2 + lax.axis_index("c")
    pltpu.sync_copy(g_hbm, g_v)
    base = wid * EPT

    def chunk(ci, carry):
        off = base + ci * CH
        pltpu.sync_copy(src_hbm.at[pl.ds(off, CH)], idx_v)

        def step(i, c2):
            for u in range(UN):
                o = i * (16 * UN) + u * 16
                s = idx_v[pl.ds(o, 16)]
                val_v[pl.ds(o, 16)] = plsc.load_gather(g_v, [s])
            return c2
        lax.fori_loop(0, CH // (16 * UN), step, 0)
        pltpu.sync_copy(val_v, msg_hbm.at[pl.ds(off, CH)])
        return carry
    lax.fori_loop(0, EPT // CH, chunk, 0)


@functools.partial(
    pl.kernel, mesh=_mesh, compiler_params=_cp,
    out_type=jax.ShapeDtypeStruct((N_TILES, N_PAD), jnp.float32),
    scratch_types=[
        pltpu.VMEM((N_PAD,), jnp.float32),
        pltpu.VMEM((CH,), jnp.int32),
        pltpu.VMEM((CH,), jnp.float32),
    ],
)
def _scatter(dst_hbm, msg_hbm, out_hbm, acc_v, idx_v, val_v):
    wid = lax.axis_index("s") * 2 + lax.axis_index("c")
    _zero_acc(acc_v)
    base = wid * EPT

    def chunk(ci, carry):
        off = base + ci * CH
        pltpu.sync_copy(dst_hbm.at[pl.ds(off, CH)], idx_v)
        pltpu.sync_copy(msg_hbm.at[pl.ds(off, CH)], val_v)

        def step(i, c2):
            for u in range(UN):
                o = i * (16 * UN) + u * 16
                d = idx_v[pl.ds(o, 16)]
                m = val_v[pl.ds(o, 16)]
                plsc.addupdate_scatter(acc_v, [d], m)
            return c2
        lax.fori_loop(0, CH // (16 * UN), step, 0)
        return carry
    lax.fori_loop(0, EPT // CH, chunk, 0)
    pltpu.sync_copy(acc_v, out_hbm.at[wid])


def _glue1_body(w_ref, degp_ref, x0_ref, x1_ref, dinv_ref, g1_ref):
    deg = jnp.sum(degp_ref[...], axis=0) + 1.0
    dinv = jax.lax.rsqrt(deg)
    z0 = x0_ref[...] * w_ref[0] + x1_ref[...] * w_ref[1]
    dinv_ref[...] = dinv
    g1_ref[...] = dinv * z0


def _glue2_body(c_ref, accp_ref, g1_ref, dinv_ref, g2_ref):
    s = jnp.sum(accp_ref[...], axis=0) + g1_ref[...]
    dinv = dinv_ref[...]
    z1 = dinv * s + c_ref[0]
    g2_ref[...] = dinv * z1


def _glue3_body(b_ref, accp_ref, g2_ref, dinv_ref, out_ref):
    s = jnp.sum(accp_ref[...], axis=0) + g2_ref[...]
    out_ref[...] = dinv_ref[...] * s + b_ref[0]


_part_spec = pl.BlockSpec((N_TILES, BK), lambda i: (0, i))
_vec_spec = pl.BlockSpec((BK,), lambda i: (i,))
_smem_spec = pl.BlockSpec(memory_space=pltpu.SMEM)
_f1d = jax.ShapeDtypeStruct((N_PAD,), jnp.float32)

_glue1 = pl.pallas_call(
    _glue1_body, grid=(N_PAD // BK,),
    in_specs=[_smem_spec, _part_spec, _vec_spec, _vec_spec],
    out_specs=[_vec_spec, _vec_spec],
    out_shape=[_f1d, _f1d],
)

_glue2 = pl.pallas_call(
    _glue2_body, grid=(N_PAD // BK,),
    in_specs=[_smem_spec, _part_spec, _vec_spec, _vec_spec],
    out_specs=[_vec_spec],
    out_shape=[_f1d],
)

_glue3 = pl.pallas_call(
    _glue3_body, grid=(N_PAD // BK,),
    in_specs=[_smem_spec, _part_spec, _vec_spec, _vec_spec],
    out_specs=[_vec_spec],
    out_shape=[_f1d],
)


def kernel(x, edge_index, W1, b1, W2, b2):
    src = edge_index[0].astype(jnp.int32)
    dst = edge_index[1].astype(jnp.int32)
    w = (W1 @ W2)[:, 0]
    c = jnp.dot(b1, W2[:, 0])

    xp = jnp.pad(x, ((0, N_PAD - N_NODES), (0, 0)))
    x0 = xp[:, 0]
    x1 = xp[:, 1]

    degp = _hist(dst)
    dinv, g1 = _glue1(w, degp, x0, x1)

    msg1 = _gather(g1, src)
    accp1 = _scatter(dst, msg1)
    (g2,) = _glue2(jnp.reshape(c, (1,)), accp1, g1, dinv)

    msg2 = _gather(g2, src)
    accp2 = _scatter(dst, msg2)
    (out,) = _glue3(b2, accp2, g2, dinv)

    return out[:N_NODES]


# speedup vs baseline: 272.9764x; 1.2019x over previous
import functools
import jax
import jax.numpy as jnp
from jax import lax
from jax.experimental import pallas as pl
from jax.experimental.pallas import tpu as pltpu
from jax.experimental.pallas import tpu_sc as plsc

N_NODES = 100000
N_PAD = 102400
N_TILES = 32
E = 3_200_000
EPT = E // N_TILES
UN = 25
BK = N_PAD // 4
GP = 16
GPS = N_PAD // GP

_mesh = plsc.VectorSubcoreMesh(core_axis_name="c", subcore_axis_name="s")
_cp = pltpu.CompilerParams(needs_layout_passes=False)


def _zero_acc(acc_v):
    zero = jnp.zeros((16,), jnp.float32)

    def zbody(i, carry):
        for u in range(16):
            acc_v[pl.ds(i * 256 + u * 16, 16)] = zero
        return carry
    lax.fori_loop(0, N_PAD // 256, zbody, 0)


@functools.partial(
    pl.kernel, mesh=_mesh, compiler_params=_cp,
    out_type=jax.ShapeDtypeStruct((N_TILES, N_PAD), jnp.float32),
    scratch_types=[
        pltpu.VMEM((N_PAD,), jnp.float32),
        pltpu.VMEM((10000,), jnp.int32),
        pltpu.VMEM((10000,), jnp.int32),
        pltpu.SemaphoreType.DMA,
        pltpu.SemaphoreType.DMA,
    ],
)
def _hist(dst_hbm, out_hbm, acc_v, idx0_v, idx1_v, si0, si1):
    wid = lax.axis_index("s") * 2 + lax.axis_index("c")
    ch, nch = 10000, 10
    base = wid * EPT
    bufs = [(idx0_v, si0), (idx1_v, si1)]

    def start(ci):
        b, s = bufs[ci % 2]
        return pltpu.async_copy(dst_hbm.at[pl.ds(base + ci * ch, ch)], b, s)

    hs = {0: start(0)}
    _zero_acc(acc_v)
    ones = jnp.ones((16,), jnp.float32)
    for ci in range(nch):
        if ci + 1 < nch:
            hs[ci + 1] = start(ci + 1)
        hs[ci].wait()
        b = bufs[ci % 2][0]

        def step(i, c2, b=b):
            for u in range(UN):
                d = b[pl.ds(i * (16 * UN) + u * 16, 16)]
                plsc.addupdate_scatter(acc_v, [d], ones)
            return c2
        lax.fori_loop(0, ch // (16 * UN), step, 0)
    pltpu.sync_copy(acc_v, out_hbm.at[wid])


@functools.partial(
    pl.kernel, mesh=_mesh, compiler_params=_cp,
    out_type=jax.ShapeDtypeStruct((E,), jnp.float32),
    scratch_types=[
        pltpu.VMEM((N_PAD,), jnp.float32),
        pltpu.VMEM((4000,), jnp.int32),
        pltpu.VMEM((4000,), jnp.int32),
        pltpu.VMEM((4000,), jnp.float32),
        pltpu.VMEM((4000,), jnp.float32),
        pltpu.SemaphoreType.DMA,
        pltpu.SemaphoreType.DMA,
        pltpu.SemaphoreType.DMA,
        pltpu.SemaphoreType.DMA,
        pltpu.SemaphoreType.DMA,
    ],
)
def _gather(g_hbm, src_hbm, msg_hbm, g_v, idx0_v, idx1_v, out0_v, out1_v,
            sg, si0, si1, so0, so1):
    wid = lax.axis_index("s") * 2 + lax.axis_index("c")
    ch, nch = 4000, 25
    base = wid * EPT
    ibufs = [(idx0_v, si0), (idx1_v, si1)]
    obufs = [(out0_v, so0), (out1_v, so1)]

    def start(ci):
        b, s = ibufs[ci % 2]
        return pltpu.async_copy(src_hbm.at[pl.ds(base + ci * ch, ch)], b, s)

    hs = {0: start(0)}
    gh = []
    for k in range(GP):
        p = lax.rem(wid + k, GP) * GPS
        gh.append(pltpu.async_copy(g_hbm.at[pl.ds(p, GPS)], g_v.at[pl.ds(p, GPS)], sg))
    for h in gh:
        h.wait()
    wb = {}
    for ci in range(nch):
        if ci + 1 < nch:
            hs[ci + 1] = start(ci + 1)
        if ci >= 2:
            wb[ci - 2].wait()
        hs[ci].wait()
        b = ibufs[ci % 2][0]
        ob = obufs[ci % 2][0]

        def step(i, c2, b=b, ob=ob):
            for u in range(UN):
                o = i * (16 * UN) + u * 16
                s = b[pl.ds(o, 16)]
                ob[pl.ds(o, 16)] = plsc.load_gather(g_v, [s])
            return c2
        lax.fori_loop(0, ch // (16 * UN), step, 0)
        wb[ci] = pltpu.async_copy(
            ob, msg_hbm.at[pl.ds(base + ci * ch, ch)], obufs[ci % 2][1])
    wb[nch - 2].wait()
    wb[nch - 1].wait()


@functools.partial(
    pl.kernel, mesh=_mesh, compiler_params=_cp,
    out_type=jax.ShapeDtypeStruct((N_TILES, N_PAD), jnp.float32),
    scratch_types=[
        pltpu.VMEM((N_PAD,), jnp.float32),
        pltpu.VMEM((4000,), jnp.int32),
        pltpu.VMEM((4000,), jnp.int32),
        pltpu.VMEM((4000,), jnp.float32),
        pltpu.VMEM((4000,), jnp.float32),
        pltpu.SemaphoreType.DMA,
        pltpu.SemaphoreType.DMA,
        pltpu.SemaphoreType.DMA,
        pltpu.SemaphoreType.DMA,
    ],
)
def _scatter(dst_hbm, msg_hbm, out_hbm, acc_v, idx0_v, idx1_v, val0_v, val1_v,
             si0, si1, sv0, sv1):
    wid = lax.axis_index("s") * 2 + lax.axis_index("c")
    ch, nch = 4000, 25
    base = wid * EPT
    bufs = [(idx0_v, si0, val0_v, sv0), (idx1_v, si1, val1_v, sv1)]

    def start(ci):
        ib, s1, vb, s2 = bufs[ci % 2]
        h1 = pltpu.async_copy(dst_hbm.at[pl.ds(base + ci * ch, ch)], ib, s1)
        h2 = pltpu.async_copy(msg_hbm.at[pl.ds(base + ci * ch, ch)], vb, s2)
        return h1, h2

    hs = {0: start(0)}
    _zero_acc(acc_v)
    for ci in range(nch):
        if ci + 1 < nch:
            hs[ci + 1] = start(ci + 1)
        hs[ci][0].wait()
        hs[ci][1].wait()
        ib, vb = bufs[ci % 2][0], bufs[ci % 2][2]

        def step(i, c2, ib=ib, vb=vb):
            for u in range(UN):
                o = i * (16 * UN) + u * 16
                d = ib[pl.ds(o, 16)]
                m = vb[pl.ds(o, 16)]
                plsc.addupdate_scatter(acc_v, [d], m)
            return c2
        lax.fori_loop(0, ch // (16 * UN), step, 0)
    pltpu.sync_copy(acc_v, out_hbm.at[wid])


def _glue1_body(w_ref, degp_ref, x0_ref, x1_ref, dinv_ref, g1_ref):
    deg = jnp.sum(degp_ref[...], axis=0) + 1.0
    dinv = jax.lax.rsqrt(deg)
    z0 = x0_ref[...] * w_ref[0] + x1_ref[...] * w_ref[1]
    dinv_ref[...] = dinv
    g1_ref[...] = dinv * z0


def _glue2_body(c_ref, accp_ref, g1_ref, dinv_ref, g2_ref):
    s = jnp.sum(accp_ref[...], axis=0) + g1_ref[...]
    dinv = dinv_ref[...]
    z1 = dinv * s + c_ref[0]
    g2_ref[...] = dinv * z1


def _glue3_body(b_ref, accp_ref, g2_ref, dinv_ref, out_ref):
    s = jnp.sum(accp_ref[...], axis=0) + g2_ref[...]
    out_ref[...] = dinv_ref[...] * s + b_ref[0]


_part_spec = pl.BlockSpec((N_TILES, BK), lambda i: (0, i))
_vec_spec = pl.BlockSpec((BK,), lambda i: (i,))
_smem_spec = pl.BlockSpec(memory_space=pltpu.SMEM)
_f1d = jax.ShapeDtypeStruct((N_PAD,), jnp.float32)

_glue1 = pl.pallas_call(
    _glue1_body, grid=(N_PAD // BK,),
    in_specs=[_smem_spec, _part_spec, _vec_spec, _vec_spec],
    out_specs=[_vec_spec, _vec_spec],
    out_shape=[_f1d, _f1d],
)

_glue2 = pl.pallas_call(
    _glue2_body, grid=(N_PAD // BK,),
    in_specs=[_smem_spec, _part_spec, _vec_spec, _vec_spec],
    out_specs=[_vec_spec],
    out_shape=[_f1d],
)

_glue3 = pl.pallas_call(
    _glue3_body, grid=(N_PAD // BK,),
    in_specs=[_smem_spec, _part_spec, _vec_spec, _vec_spec],
    out_specs=[_vec_spec],
    out_shape=[_f1d],
)


def kernel(x, edge_index, W1, b1, W2, b2):
    src = edge_index[0].astype(jnp.int32)
    dst = edge_index[1].astype(jnp.int32)
    w = (W1 @ W2)[:, 0]
    c = jnp.dot(b1, W2[:, 0])

    xp = jnp.pad(x, ((0, N_PAD - N_NODES), (0, 0)))
    x0 = xp[:, 0]
    x1 = xp[:, 1]

    degp = _hist(dst)
    dinv, g1 = _glue1(w, degp, x0, x1)

    msg1 = _gather(g1, src)
    accp1 = _scatter(dst, msg1)
    (g2,) = _glue2(jnp.reshape(c, (1,)), accp1, g1, dinv)

    msg2 = _gather(g2, src)
    accp2 = _scatter(dst, msg2)
    (out,) = _glue3(b2, accp2, g2, dinv)

    return out[:N_NODES]


# speedup vs baseline: 283.0616x; 1.0369x over previous
import functools
import jax
import jax.numpy as jnp
from jax import lax
from jax.experimental import pallas as pl
from jax.experimental.pallas import tpu as pltpu
from jax.experimental.pallas import tpu_sc as plsc

N_NODES = 100000
N_PAD = 102400
N_TILES = 32
E = 3_200_000
EPT = E // N_TILES
UN = 25
BK = N_PAD // 4
GP = 16
GPS = N_PAD // GP

_mesh = plsc.VectorSubcoreMesh(core_axis_name="c", subcore_axis_name="s")
_cp = pltpu.CompilerParams(needs_layout_passes=False)


def _zero_acc(acc_v):
    zero = jnp.zeros((16,), jnp.float32)

    def zbody(i, carry):
        for u in range(16):
            acc_v[pl.ds(i * 256 + u * 16, 16)] = zero
        return carry
    lax.fori_loop(0, N_PAD // 256, zbody, 0)


@functools.partial(
    pl.kernel, mesh=_mesh, compiler_params=_cp,
    out_type=jax.ShapeDtypeStruct((N_TILES, N_PAD), jnp.float32),
    scratch_types=[
        pltpu.VMEM((N_PAD,), jnp.float32),
        pltpu.VMEM((10000,), jnp.int32),
        pltpu.VMEM((10000,), jnp.int32),
        pltpu.SemaphoreType.DMA,
        pltpu.SemaphoreType.DMA,
    ],
)
def _hist(dst_hbm, out_hbm, acc_v, idx0_v, idx1_v, si0, si1):
    wid = lax.axis_index("s") * 2 + lax.axis_index("c")
    ch, nch = 10000, 10
    base = wid * EPT
    bufs = [(idx0_v, si0), (idx1_v, si1)]

    def start(ci):
        b, s = bufs[ci % 2]
        return pltpu.async_copy(dst_hbm.at[pl.ds(base + ci * ch, ch)], b, s)

    hs = {0: start(0)}
    _zero_acc(acc_v)
    ones = jnp.ones((16,), jnp.float32)
    for ci in range(nch):
        if ci + 1 < nch:
            hs[ci + 1] = start(ci + 1)
        hs[ci].wait()
        b = bufs[ci % 2][0]

        def step(i, c2, b=b):
            for u in range(UN):
                d = b[pl.ds(i * (16 * UN) + u * 16, 16)]
                plsc.addupdate_scatter(acc_v, [d], ones)
            return c2
        lax.fori_loop(0, ch // (16 * UN), step, 0)
    pltpu.sync_copy(acc_v, out_hbm.at[wid])


@functools.partial(
    pl.kernel, mesh=_mesh, compiler_params=_cp,
    out_type=(
        jax.ShapeDtypeStruct((N_TILES, N_PAD), jnp.float32),
        jax.ShapeDtypeStruct((E,), jnp.float32),
    ),
    scratch_types=[
        pltpu.VMEM((N_PAD,), jnp.float32),
        pltpu.VMEM((4000,), jnp.int32),
        pltpu.VMEM((4000,), jnp.int32),
        pltpu.VMEM((4000,), jnp.float32),
        pltpu.VMEM((4000,), jnp.float32),
        pltpu.SemaphoreType.DMA,
        pltpu.SemaphoreType.DMA,
        pltpu.SemaphoreType.DMA,
        pltpu.SemaphoreType.DMA,
        pltpu.SemaphoreType.DMA,
    ],
)
def _round(g_hbm, src_hbm, dst_hbm, out_hbm, msg_hbm, ga_v,
           idx0_v, idx1_v, val0_v, val1_v, sg, si0, si1, sv0, sv1):
    wid = lax.axis_index("s") * 2 + lax.axis_index("c")
    ch, nch = 4000, 25
    base = wid * EPT
    ibufs = [(idx0_v, si0), (idx1_v, si1)]
    vbufs = [(val0_v, sv0), (val1_v, sv1)]

    def start_src(ci):
        b, s = ibufs[ci % 2]
        return pltpu.async_copy(src_hbm.at[pl.ds(base + ci * ch, ch)], b, s)

    hs = {0: start_src(0)}
    gh = []
    for k in range(GP):
        p = lax.rem(wid + k, GP) * GPS
        gh.append(pltpu.async_copy(
            g_hbm.at[pl.ds(p, GPS)], ga_v.at[pl.ds(p, GPS)], sg))
    for h in gh:
        h.wait()
    wb = {}
    for ci in range(nch):
        if ci + 1 < nch:
            hs[ci + 1] = start_src(ci + 1)
        if ci >= 2:
            wb[ci - 2].wait()
        hs[ci].wait()
        b = ibufs[ci % 2][0]
        ob = vbufs[ci % 2][0]

        def gstep(i, c2, b=b, ob=ob):
            for u in range(UN):
                o = i * (16 * UN) + u * 16
                s = b[pl.ds(o, 16)]
                ob[pl.ds(o, 16)] = plsc.load_gather(ga_v, [s])
            return c2
        lax.fori_loop(0, ch // (16 * UN), gstep, 0)
        wb[ci] = pltpu.async_copy(
            ob, msg_hbm.at[pl.ds(base + ci * ch, ch)], vbufs[ci % 2][1])
    wb[nch - 2].wait()
    wb[nch - 1].wait()

    def start_pair(ci):
        ib, s1 = ibufs[ci % 2]
        vb, s2 = vbufs[ci % 2]
        h1 = pltpu.async_copy(dst_hbm.at[pl.ds(base + ci * ch, ch)], ib, s1)
        h2 = pltpu.async_copy(msg_hbm.at[pl.ds(base + ci * ch, ch)], vb, s2)
        return h1, h2

    ps = {0: start_pair(0)}
    _zero_acc(ga_v)
    for ci in range(nch):
        if ci + 1 < nch:
            ps[ci + 1] = start_pair(ci + 1)
        ps[ci][0].wait()
        ps[ci][1].wait()
        ib = ibufs[ci % 2][0]
        vb = vbufs[ci % 2][0]

        def sstep(i, c2, ib=ib, vb=vb):
            for u in range(UN):
                o = i * (16 * UN) + u * 16
                d = ib[pl.ds(o, 16)]
                m = vb[pl.ds(o, 16)]
                plsc.addupdate_scatter(ga_v, [d], m)
            return c2
        lax.fori_loop(0, ch // (16 * UN), sstep, 0)
    pltpu.sync_copy(ga_v, out_hbm.at[wid])


def _glue1_body(w_ref, degp_ref, x0_ref, x1_ref, dinv_ref, g1_ref):
    deg = jnp.sum(degp_ref[...], axis=0) + 1.0
    dinv = jax.lax.rsqrt(deg)
    z0 = x0_ref[...] * w_ref[0] + x1_ref[...] * w_ref[1]
    dinv_ref[...] = dinv
    g1_ref[...] = dinv * z0


def _glue2_body(c_ref, accp_ref, g1_ref, dinv_ref, g2_ref):
    s = jnp.sum(accp_ref[...], axis=0) + g1_ref[...]
    dinv = dinv_ref[...]
    z1 = dinv * s + c_ref[0]
    g2_ref[...] = dinv * z1


def _glue3_body(b_ref, accp_ref, g2_ref, dinv_ref, out_ref):
    s = jnp.sum(accp_ref[...], axis=0) + g2_ref[...]
    out_ref[...] = dinv_ref[...] * s + b_ref[0]


_part_spec = pl.BlockSpec((N_TILES, BK), lambda i: (0, i))
_vec_spec = pl.BlockSpec((BK,), lambda i: (i,))
_smem_spec = pl.BlockSpec(memory_space=pltpu.SMEM)
_f1d = jax.ShapeDtypeStruct((N_PAD,), jnp.float32)

_glue1 = pl.pallas_call(
    _glue1_body, grid=(N_PAD // BK,),
    in_specs=[_smem_spec, _part_spec, _vec_spec, _vec_spec],
    out_specs=[_vec_spec, _vec_spec],
    out_shape=[_f1d, _f1d],
)

_glue2 = pl.pallas_call(
    _glue2_body, grid=(N_PAD // BK,),
    in_specs=[_smem_spec, _part_spec, _vec_spec, _vec_spec],
    out_specs=[_vec_spec],
    out_shape=[_f1d],
)

_glue3 = pl.pallas_call(
    _glue3_body, grid=(N_PAD // BK,),
    in_specs=[_smem_spec, _part_spec, _vec_spec, _vec_spec],
    out_specs=[_vec_spec],
    out_shape=[_f1d],
)


def kernel(x, edge_index, W1, b1, W2, b2):
    src = edge_index[0].astype(jnp.int32)
    dst = edge_index[1].astype(jnp.int32)
    w = (W1 @ W2)[:, 0]
    c = jnp.dot(b1, W2[:, 0])

    xp = jnp.pad(x, ((0, N_PAD - N_NODES), (0, 0)))
    x0 = xp[:, 0]
    x1 = xp[:, 1]

    degp = _hist(dst)
    dinv, g1 = _glue1(w, degp, x0, x1)

    accp1, _ = _round(g1, src, dst)
    (g2,) = _glue2(jnp.reshape(c, (1,)), accp1, g1, dinv)

    accp2, _ = _round(g2, src, dst)
    (out,) = _glue3(b2, accp2, g2, dinv)

    return out[:N_NODES]


# speedup vs baseline: 455.0216x; 1.6075x over previous
import functools
import jax
import jax.numpy as jnp
from jax import lax
from jax.experimental import pallas as pl
from jax.experimental.pallas import tpu as pltpu
from jax.experimental.pallas import tpu_sc as plsc

N_NODES = 100000
N_PAD = 102400
N_TILES = 32
E = 3_200_000
E_PAD = 3_276_800
EPT = E_PAD // N_TILES
BK = N_PAD // 4
GP = 16
GPS = N_PAD // GP
NSL = N_PAD // 16

_mesh = plsc.VectorSubcoreMesh(core_axis_name="c", subcore_axis_name="s")
_cp = pltpu.CompilerParams(needs_layout_passes=False)


@functools.partial(
    pl.kernel, mesh=_mesh, compiler_params=_cp,
    out_type=jax.ShapeDtypeStruct((N_TILES, N_PAD), jnp.float32),
    scratch_types=[
        pltpu.VMEM((N_PAD,), jnp.float32),
        pltpu.VMEM((10240,), jnp.int32),
        pltpu.VMEM((10240,), jnp.int32),
        pltpu.SemaphoreType.DMA,
        pltpu.SemaphoreType.DMA,
    ],
)
def _hist(dst_hbm, out_hbm, acc_v, idx0_v, idx1_v, si0, si1):
    wid = lax.axis_index("s") * 2 + lax.axis_index("c")
    ch, nch, un = 10240, 10, 32
    base = wid * EPT
    bufs = [(idx0_v, si0), (idx1_v, si1)]

    def start(ci):
        b, s = bufs[ci % 2]
        return pltpu.async_copy(dst_hbm.at[pl.ds(base + ci * ch, ch)], b, s)

    hs = {0: start(0)}
    zero = jnp.zeros((16,), jnp.float32)

    def zbody(i, carry):
        for u in range(16):
            acc_v[pl.ds(i * 256 + u * 16, 16)] = zero
        return carry
    lax.fori_loop(0, N_PAD // 256, zbody, 0)

    ones = jnp.ones((16,), jnp.float32)
    for ci in range(nch):
        if ci + 1 < nch:
            hs[ci + 1] = start(ci + 1)
        hs[ci].wait()
        b = bufs[ci % 2][0]

        def step(i, c2, b=b):
            for u in range(un):
                d = b[pl.ds(i * (16 * un) + u * 16, 16)]
                plsc.addupdate_scatter(acc_v, [d], ones)
            return c2
        lax.fori_loop(0, ch // (16 * un), step, 0)
    pltpu.sync_copy(acc_v, out_hbm.at[wid])


@functools.partial(
    pl.kernel, mesh=_mesh, compiler_params=_cp,
    out_type=jax.ShapeDtypeStruct((2, N_PAD), jnp.float32),
    scratch_types=[
        pltpu.VMEM_SHARED((N_PAD,), jnp.float32),
        pltpu.VMEM((N_PAD,), jnp.float32),
        pltpu.VMEM((2048,), jnp.int32),
        pltpu.VMEM((2048,), jnp.int32),
        pltpu.VMEM((2048,), jnp.int32),
        pltpu.VMEM((2048,), jnp.int32),
        pltpu.VMEM((2048,), jnp.int32),
        pltpu.VMEM((2048,), jnp.int32),
        pltpu.VMEM((2048,), jnp.float32),
        pltpu.VMEM((2048,), jnp.float32),
        pltpu.VMEM((2048,), jnp.float32),
        pltpu.SemaphoreType.DMA,
        pltpu.SemaphoreType.DMA,
        pltpu.SemaphoreType.DMA,
        pltpu.SemaphoreType.DMA,
        pltpu.SemaphoreType.DMA,
        pltpu.SemaphoreType.DMA,
        pltpu.SemaphoreType.DMA,
        pltpu.SemaphoreType.DMA,
        pltpu.SemaphoreType.DMA,
        pltpu.SemaphoreType.DMA,
    ],
)
def _round(g_hbm, src_hbm, dst_hbm, out_hbm, acc_sp, g_v,
           src0_v, src1_v, src2_v, dst0_v, dst1_v, dst2_v,
           val0_v, val1_v, val2_v,
           sg, ss0, ss1, ss2, sd0, sd1, sd2, sv0, sv1, sv2):
    cid = lax.axis_index("c")
    sid = lax.axis_index("s")
    wid = sid * 2 + cid
    ch, nch, un = 2048, 50, 8
    base = wid * EPT
    sbufs = [(src0_v, ss0), (src1_v, ss1), (src2_v, ss2)]
    dbufs = [(dst0_v, sd0), (dst1_v, sd1), (dst2_v, sd2)]
    vbufs = [(val0_v, sv0), (val1_v, sv1), (val2_v, sv2)]

    def start_src(ci):
        b, s = sbufs[ci % 3]
        return pltpu.async_copy(src_hbm.at[pl.ds(base + ci * ch, ch)], b, s)

    def start_dst(ci):
        b, s = dbufs[ci % 3]
        return pltpu.async_copy(dst_hbm.at[pl.ds(base + ci * ch, ch)], b, s)

    hs = {0: (start_src(0), start_dst(0))}

    zero = jnp.zeros((16,), jnp.float32)

    def zbody(i, carry):
        for u in range(16):
            g_v[pl.ds(i * 256 + u * 16, 16)] = zero
        return carry
    lax.fori_loop(0, NSL // 256, zbody, 0)
    pltpu.sync_copy(g_v.at[pl.ds(0, NSL)], acc_sp.at[pl.ds(sid * NSL, NSL)])
    plsc.subcore_barrier()

    gh = []
    for k in range(GP):
        p = lax.rem(wid + k, GP) * GPS
        gh.append(pltpu.async_copy(
            g_hbm.at[pl.ds(p, GPS)], g_v.at[pl.ds(p, GPS)], sg))
    for h in gh:
        h.wait()

    wb = {}
    for ci in range(nch):
        if ci >= 2:
            wb[ci - 2].wait()
        if ci + 1 < nch:
            hs[ci + 1] = (start_src(ci + 1), start_dst(ci + 1))
        hs[ci][0].wait()
        b = sbufs[ci % 3][0]
        ob = vbufs[ci % 3][0]

        def gstep(i, c2, b=b, ob=ob):
            for u in range(un):
                o = i * (16 * un) + u * 16
                s = b[pl.ds(o, 16)]
                ob[pl.ds(o, 16)] = plsc.load_gather(g_v, [s])
            return c2
        lax.fori_loop(0, ch // (16 * un), gstep, 0)
        hs[ci][1].wait()
        wb[ci] = pltpu.async_copy(
            ob, acc_sp.at[dbufs[ci % 3][0]], vbufs[ci % 3][1], add=True)
    wb[nch - 2].wait()
    wb[nch - 1].wait()
    plsc.subcore_barrier()
    pltpu.sync_copy(acc_sp.at[pl.ds(sid * NSL, NSL)],
                    out_hbm.at[cid, pl.ds(sid * NSL, NSL)])


def _glue1_body(w_ref, degp_ref, x0_ref, x1_ref, dinv_ref, g1_ref):
    deg = jnp.sum(degp_ref[...], axis=0) + 1.0
    dinv = jax.lax.rsqrt(deg)
    z0 = x0_ref[...] * w_ref[0] + x1_ref[...] * w_ref[1]
    dinv_ref[...] = dinv
    g1_ref[...] = dinv * z0


def _glue2_body(c_ref, accp_ref, g1_ref, dinv_ref, g2_ref):
    s = jnp.sum(accp_ref[...], axis=0) + g1_ref[...]
    dinv = dinv_ref[...]
    z1 = dinv * s + c_ref[0]
    g2_ref[...] = dinv * z1


def _glue3_body(b_ref, accp_ref, g2_ref, dinv_ref, out_ref):
    s = jnp.sum(accp_ref[...], axis=0) + g2_ref[...]
    out_ref[...] = dinv_ref[...] * s + b_ref[0]


_part_spec = pl.BlockSpec((N_TILES, BK), lambda i: (0, i))
_part2_spec = pl.BlockSpec((2, BK), lambda i: (0, i))
_vec_spec = pl.BlockSpec((BK,), lambda i: (i,))
_smem_spec = pl.BlockSpec(memory_space=pltpu.SMEM)
_f1d = jax.ShapeDtypeStruct((N_PAD,), jnp.float32)

_glue1 = pl.pallas_call(
    _glue1_body, grid=(N_PAD // BK,),
    in_specs=[_smem_spec, _part_spec, _vec_spec, _vec_spec],
    out_specs=[_vec_spec, _vec_spec],
    out_shape=[_f1d, _f1d],
)

_glue2 = pl.pallas_call(
    _glue2_body, grid=(N_PAD // BK,),
    in_specs=[_smem_spec, _part2_spec, _vec_spec, _vec_spec],
    out_specs=[_vec_spec],
    out_shape=[_f1d],
)

_glue3 = pl.pallas_call(
    _glue3_body, grid=(N_PAD // BK,),
    in_specs=[_smem_spec, _part2_spec, _vec_spec, _vec_spec],
    out_specs=[_vec_spec],
    out_shape=[_f1d],
)


def kernel(x, edge_index, W1, b1, W2, b2):
    src = edge_index[0].astype(jnp.int32)
    dst = edge_index[1].astype(jnp.int32)
    npad = E_PAD - E
    src_p = jnp.concatenate([src, jnp.zeros((npad,), jnp.int32)])
    dst_p = jnp.concatenate(
        [dst, (jnp.arange(npad, dtype=jnp.int32) % 2048) + (N_NODES + 352)])

    w = (W1 @ W2)[:, 0]
    c = jnp.dot(b1, W2[:, 0])

    xp = jnp.pad(x, ((0, N_PAD - N_NODES), (0, 0)))
    x0 = xp[:, 0]
    x1 = xp[:, 1]

    degp = _hist(dst_p)
    dinv, g1 = _glue1(w, degp, x0, x1)

    accp1 = _round(g1, src_p, dst_p)
    (g2,) = _glue2(jnp.reshape(c, (1,)), accp1, g1, dinv)

    accp2 = _round(g2, src_p, dst_p)
    (out,) = _glue3(b2, accp2, g2, dinv)

    return out[:N_NODES]


# speedup vs baseline: 476.9305x; 1.0481x over previous
import functools
import jax
import jax.numpy as jnp
from jax import lax
from jax.experimental import pallas as pl
from jax.experimental.pallas import tpu as pltpu
from jax.experimental.pallas import tpu_sc as plsc

N_NODES = 100000
N_PAD = 102400
N_TILES = 32
E = 3_200_000
EPT = E // N_TILES
BK = N_PAD // 4
GP = 16
GPS = N_PAD // GP
NSL = N_PAD // 16

_mesh = plsc.VectorSubcoreMesh(core_axis_name="c", subcore_axis_name="s")
_cp = pltpu.CompilerParams(needs_layout_passes=False)


@functools.partial(
    pl.kernel, mesh=_mesh, compiler_params=_cp,
    out_type=jax.ShapeDtypeStruct((N_TILES, N_PAD), jnp.float32),
    scratch_types=[
        pltpu.VMEM((N_PAD,), jnp.float32),
        pltpu.VMEM((10000,), jnp.int32),
        pltpu.VMEM((10000,), jnp.int32),
        pltpu.SemaphoreType.DMA,
        pltpu.SemaphoreType.DMA,
    ],
)
def _hist(dst_hbm, out_hbm, acc_v, idx0_v, idx1_v, si0, si1):
    wid = lax.axis_index("s") * 2 + lax.axis_index("c")
    ch, nch, un = 10000, 10, 25
    base = wid * EPT
    bufs = [(idx0_v, si0), (idx1_v, si1)]

    def start(ci):
        b, s = bufs[ci % 2]
        return pltpu.async_copy(dst_hbm.at[pl.ds(base + ci * ch, ch)], b, s)

    hs = {0: start(0)}
    zero = jnp.zeros((16,), jnp.float32)

    def zbody(i, carry):
        for u in range(16):
            acc_v[pl.ds(i * 256 + u * 16, 16)] = zero
        return carry
    lax.fori_loop(0, N_PAD // 256, zbody, 0)

    ones = jnp.ones((16,), jnp.float32)
    for ci in range(nch):
        if ci + 1 < nch:
            hs[ci + 1] = start(ci + 1)
        hs[ci].wait()
        b = bufs[ci % 2][0]

        def step(i, c2, b=b):
            for u in range(un):
                d = b[pl.ds(i * (16 * un) + u * 16, 16)]
                plsc.addupdate_scatter(acc_v, [d], ones)
            return c2
        lax.fori_loop(0, ch // (16 * un), step, 0)
    pltpu.sync_copy(acc_v, out_hbm.at[wid])


@functools.partial(
    pl.kernel, mesh=_mesh, compiler_params=_cp,
    out_type=jax.ShapeDtypeStruct((2, N_PAD), jnp.float32),
    scratch_types=[
        pltpu.VMEM_SHARED((N_PAD,), jnp.float32),
        pltpu.VMEM((N_PAD,), jnp.float32),
        pltpu.VMEM((2000,), jnp.int32),
        pltpu.VMEM((2000,), jnp.int32),
        pltpu.VMEM((2000,), jnp.int32),
        pltpu.VMEM((2000,), jnp.int32),
        pltpu.VMEM((2000,), jnp.int32),
        pltpu.VMEM((2000,), jnp.int32),
        pltpu.VMEM((2000,), jnp.float32),
        pltpu.VMEM((2000,), jnp.float32),
        pltpu.VMEM((2000,), jnp.float32),
        pltpu.SemaphoreType.DMA,
        pltpu.SemaphoreType.DMA,
        pltpu.SemaphoreType.DMA,
        pltpu.SemaphoreType.DMA,
        pltpu.SemaphoreType.DMA,
        pltpu.SemaphoreType.DMA,
        pltpu.SemaphoreType.DMA,
        pltpu.SemaphoreType.DMA,
        pltpu.SemaphoreType.DMA,
        pltpu.SemaphoreType.DMA,
    ],
)
def _round(g_hbm, src_hbm, dst_hbm, out_hbm, acc_sp, g_v,
           src0_v, src1_v, src2_v, dst0_v, dst1_v, dst2_v,
           val0_v, val1_v, val2_v,
           sg, ss0, ss1, ss2, sd0, sd1, sd2, sv0, sv1, sv2):
    cid = lax.axis_index("c")
    sid = lax.axis_index("s")
    wid = sid * 2 + cid
    ch, nch, un = 2000, 50, 5
    base = wid * EPT
    sbufs = [(src0_v, ss0), (src1_v, ss1), (src2_v, ss2)]
    dbufs = [(dst0_v, sd0), (dst1_v, sd1), (dst2_v, sd2)]
    vbufs = [(val0_v, sv0), (val1_v, sv1), (val2_v, sv2)]

    def start_src(ci):
        b, s = sbufs[ci % 3]
        return pltpu.async_copy(src_hbm.at[pl.ds(base + ci * ch, ch)], b, s)

    def start_dst(ci):
        b, s = dbufs[ci % 3]
        return pltpu.async_copy(dst_hbm.at[pl.ds(base + ci * ch, ch)], b, s)

    hs = {0: (start_src(0), start_dst(0))}

    zero = jnp.zeros((16,), jnp.float32)

    def zbody(i, carry):
        for u in range(16):
            g_v[pl.ds(i * 256 + u * 16, 16)] = zero
        return carry
    lax.fori_loop(0, NSL // 256, zbody, 0)
    pltpu.sync_copy(g_v.at[pl.ds(0, NSL)], acc_sp.at[pl.ds(sid * NSL, NSL)])
    plsc.subcore_barrier()

    gh = []
    for k in range(GP):
        p = lax.rem(wid + k, GP) * GPS
        gh.append(pltpu.async_copy(
            g_hbm.at[pl.ds(p, GPS)], g_v.at[pl.ds(p, GPS)], sg))
    for h in gh:
        h.wait()

    wb = {}
    for ci in range(nch):
        if ci >= 2:
            wb[ci - 2].wait()
        if ci + 1 < nch:
            hs[ci + 1] = (start_src(ci + 1), start_dst(ci + 1))
        hs[ci][0].wait()
        b = sbufs[ci % 3][0]
        ob = vbufs[ci % 3][0]

        def gstep(i, c2, b=b, ob=ob):
            for u in range(un):
                o = i * (16 * un) + u * 16
                s = b[pl.ds(o, 16)]
                ob[pl.ds(o, 16)] = plsc.load_gather(g_v, [s])
            return c2
        lax.fori_loop(0, ch // (16 * un), gstep, 0)
        hs[ci][1].wait()
        wb[ci] = pltpu.async_copy(
            ob, acc_sp.at[dbufs[ci % 3][0]], vbufs[ci % 3][1], add=True)
    wb[nch - 2].wait()
    wb[nch - 1].wait()
    plsc.subcore_barrier()
    pltpu.sync_copy(acc_sp.at[pl.ds(sid * NSL, NSL)],
                    out_hbm.at[cid, pl.ds(sid * NSL, NSL)])


def _glue1_body(w_ref, degp_ref, x0_ref, x1_ref, dinv_ref, g1_ref):
    deg = jnp.sum(degp_ref[...], axis=0) + 1.0
    dinv = jax.lax.rsqrt(deg)
    z0 = x0_ref[...] * w_ref[0] + x1_ref[...] * w_ref[1]
    dinv_ref[...] = dinv
    g1_ref[...] = dinv * z0


def _glue2_body(c_ref, accp_ref, g1_ref, dinv_ref, g2_ref):
    s = jnp.sum(accp_ref[...], axis=0) + g1_ref[...]
    dinv = dinv_ref[...]
    z1 = dinv * s + c_ref[0]
    g2_ref[...] = dinv * z1


def _glue3_body(b_ref, accp_ref, g2_ref, dinv_ref, out_ref):
    s = jnp.sum(accp_ref[...], axis=0) + g2_ref[...]
    out_ref[...] = dinv_ref[...] * s + b_ref[0]


_part_spec = pl.BlockSpec((N_TILES, BK), lambda i: (0, i))
_part2_spec = pl.BlockSpec((2, BK), lambda i: (0, i))
_vec_spec = pl.BlockSpec((BK,), lambda i: (i,))
_smem_spec = pl.BlockSpec(memory_space=pltpu.SMEM)
_f1d = jax.ShapeDtypeStruct((N_PAD,), jnp.float32)

_glue1 = pl.pallas_call(
    _glue1_body, grid=(N_PAD // BK,),
    in_specs=[_smem_spec, _part_spec, _vec_spec, _vec_spec],
    out_specs=[_vec_spec, _vec_spec],
    out_shape=[_f1d, _f1d],
)

_glue2 = pl.pallas_call(
    _glue2_body, grid=(N_PAD // BK,),
    in_specs=[_smem_spec, _part2_spec, _vec_spec, _vec_spec],
    out_specs=[_vec_spec],
    out_shape=[_f1d],
)

_glue3 = pl.pallas_call(
    _glue3_body, grid=(N_PAD // BK,),
    in_specs=[_smem_spec, _part2_spec, _vec_spec, _vec_spec],
    out_specs=[_vec_spec],
    out_shape=[_f1d],
)


def kernel(x, edge_index, W1, b1, W2, b2):
    src = edge_index[0].astype(jnp.int32)
    dst = edge_index[1].astype(jnp.int32)

    w = (W1 @ W2)[:, 0]
    c = jnp.dot(b1, W2[:, 0])

    xp = jnp.pad(x, ((0, N_PAD - N_NODES), (0, 0)))
    x0 = xp[:, 0]
    x1 = xp[:, 1]

    degp = _hist(dst)
    dinv, g1 = _glue1(w, degp, x0, x1)

    accp1 = _round(g1, src, dst)
    (g2,) = _glue2(jnp.reshape(c, (1,)), accp1, g1, dinv)

    accp2 = _round(g2, src, dst)
    (out,) = _glue3(b2, accp2, g2, dinv)

    return out[:N_NODES]


# speedup vs baseline: 482.3055x; 1.0113x over previous
import functools
import jax
import jax.numpy as jnp
from jax import lax
from jax.experimental import pallas as pl
from jax.experimental.pallas import tpu as pltpu
from jax.experimental.pallas import tpu_sc as plsc

N_NODES = 100000
N_PAD = 102400
N_TILES = 32
E = 3_200_000
EPT = E // N_TILES
BK = N_PAD // 4
GP = 16
GPS = N_PAD // GP
NSL = N_PAD // 16

_mesh = plsc.VectorSubcoreMesh(core_axis_name="c", subcore_axis_name="s")
_cp = pltpu.CompilerParams(needs_layout_passes=False)


@functools.partial(
    pl.kernel, mesh=_mesh, compiler_params=_cp,
    out_type=(
        jax.ShapeDtypeStruct((N_TILES, N_PAD), jnp.float32),
        jax.ShapeDtypeStruct((2, N_PAD), jnp.float32),
    ),
    scratch_types=[
        pltpu.VMEM_SHARED((N_PAD,), jnp.float32),
        pltpu.VMEM((N_PAD,), jnp.float32),
        pltpu.VMEM((2000,), jnp.int32),
        pltpu.VMEM((2000,), jnp.int32),
        pltpu.VMEM((2000,), jnp.int32),
        pltpu.VMEM((2000,), jnp.float32),
        pltpu.SemaphoreType.DMA,
        pltpu.SemaphoreType.DMA,
        pltpu.SemaphoreType.DMA,
        pltpu.SemaphoreType.DMA,
        pltpu.SemaphoreType.DMA,
        pltpu.SemaphoreType.DMA,
    ],
)
def _hist(dst_hbm, out_hbm, outsp_hbm, acc_sp, acc_v,
          idx0_v, idx1_v, idx2_v, ones_v, si0, si1, si2, sw0, sw1, sw2):
    cid = lax.axis_index("c")
    sid = lax.axis_index("s")
    wid = sid * 2 + cid
    ch, nch, un = 2000, 50, 5
    ntec = 25
    base = wid * EPT
    bufs = [(idx0_v, si0), (idx1_v, si1), (idx2_v, si2)]
    wsems = [sw0, sw1, sw2]

    def start(ci):
        b, s = bufs[ci % 3]
        return pltpu.async_copy(dst_hbm.at[pl.ds(base + ci * ch, ch)], b, s)

    hs = {0: start(0)}
    zero = jnp.zeros((16,), jnp.float32)

    def zbody(i, carry):
        for u in range(16):
            acc_v[pl.ds(i * 256 + u * 16, 16)] = zero
        return carry
    lax.fori_loop(0, N_PAD // 256, zbody, 0)

    ones = jnp.ones((16,), jnp.float32)

    def obody(i, carry):
        for u in range(5):
            ones_v[pl.ds(i * 80 + u * 16, 16)] = ones
        return carry
    lax.fori_loop(0, ch // 80, obody, 0)
    pltpu.sync_copy(acc_v.at[pl.ds(0, NSL)], acc_sp.at[pl.ds(sid * NSL, NSL)])
    plsc.subcore_barrier()

    wb = {}
    for ci in range(nch):
        if ci - 2 in wb:
            wb[ci - 2].wait()
        if ci + 1 < nch:
            hs[ci + 1] = start(ci + 1)
        hs[ci].wait()
        b = bufs[ci % 3][0]
        if ci < ntec:
            def step(i, c2, b=b):
                for u in range(un):
                    d = b[pl.ds(i * (16 * un) + u * 16, 16)]
                    plsc.addupdate_scatter(acc_v, [d], ones)
                return c2
            lax.fori_loop(0, ch // (16 * un), step, 0)
        else:
            wb[ci] = pltpu.async_copy(
                ones_v, acc_sp.at[b], wsems[ci % 3], add=True)
    wb[nch - 2].wait()
    wb[nch - 1].wait()
    pltpu.sync_copy(acc_v, out_hbm.at[wid])
    plsc.subcore_barrier()
    pltpu.sync_copy(acc_sp.at[pl.ds(sid * NSL, NSL)],
                    outsp_hbm.at[cid, pl.ds(sid * NSL, NSL)])


@functools.partial(
    pl.kernel, mesh=_mesh, compiler_params=_cp,
    out_type=jax.ShapeDtypeStruct((2, N_PAD), jnp.float32),
    scratch_types=[
        pltpu.VMEM_SHARED((N_PAD,), jnp.float32),
        pltpu.VMEM((N_PAD,), jnp.float32),
        pltpu.VMEM((2000,), jnp.int32),
        pltpu.VMEM((2000,), jnp.int32),
        pltpu.VMEM((2000,), jnp.int32),
        pltpu.VMEM((2000,), jnp.int32),
        pltpu.VMEM((2000,), jnp.int32),
        pltpu.VMEM((2000,), jnp.int32),
        pltpu.VMEM((2000,), jnp.float32),
        pltpu.VMEM((2000,), jnp.float32),
        pltpu.VMEM((2000,), jnp.float32),
        pltpu.SemaphoreType.DMA,
        pltpu.SemaphoreType.DMA,
        pltpu.SemaphoreType.DMA,
        pltpu.SemaphoreType.DMA,
        pltpu.SemaphoreType.DMA,
        pltpu.SemaphoreType.DMA,
        pltpu.SemaphoreType.DMA,
        pltpu.SemaphoreType.DMA,
        pltpu.SemaphoreType.DMA,
        pltpu.SemaphoreType.DMA,
    ],
)
def _round(g_hbm, src_hbm, dst_hbm, out_hbm, acc_sp, g_v,
           src0_v, src1_v, src2_v, dst0_v, dst1_v, dst2_v,
           val0_v, val1_v, val2_v,
           sg, ss0, ss1, ss2, sd0, sd1, sd2, sv0, sv1, sv2):
    cid = lax.axis_index("c")
    sid = lax.axis_index("s")
    wid = sid * 2 + cid
    ch, nch, un = 2000, 50, 5
    base = wid * EPT
    sbufs = [(src0_v, ss0), (src1_v, ss1), (src2_v, ss2)]
    dbufs = [(dst0_v, sd0), (dst1_v, sd1), (dst2_v, sd2)]
    vbufs = [(val0_v, sv0), (val1_v, sv1), (val2_v, sv2)]

    def start_src(ci):
        b, s = sbufs[ci % 3]
        return pltpu.async_copy(src_hbm.at[pl.ds(base + ci * ch, ch)], b, s)

    def start_dst(ci):
        b, s = dbufs[ci % 3]
        return pltpu.async_copy(dst_hbm.at[pl.ds(base + ci * ch, ch)], b, s)

    hs = {0: (start_src(0), start_dst(0))}

    zero = jnp.zeros((16,), jnp.float32)

    def zbody(i, carry):
        for u in range(16):
            g_v[pl.ds(i * 256 + u * 16, 16)] = zero
        return carry
    lax.fori_loop(0, NSL // 256, zbody, 0)
    pltpu.sync_copy(g_v.at[pl.ds(0, NSL)], acc_sp.at[pl.ds(sid * NSL, NSL)])
    plsc.subcore_barrier()

    gh = []
    for k in range(GP):
        p = lax.rem(wid + k, GP) * GPS
        gh.append(pltpu.async_copy(
            g_hbm.at[pl.ds(p, GPS)], g_v.at[pl.ds(p, GPS)], sg))
    for h in gh:
        h.wait()

    wb = {}
    for ci in range(nch):
        if ci >= 2:
            wb[ci - 2].wait()
        if ci + 1 < nch:
            hs[ci + 1] = (start_src(ci + 1), start_dst(ci + 1))
        hs[ci][0].wait()
        b = sbufs[ci % 3][0]
        ob = vbufs[ci % 3][0]

        def gstep(i, c2, b=b, ob=ob):
            for u in range(un):
                o = i * (16 * un) + u * 16
                s = b[pl.ds(o, 16)]
                ob[pl.ds(o, 16)] = plsc.load_gather(g_v, [s])
            return c2
        lax.fori_loop(0, ch // (16 * un), gstep, 0)
        hs[ci][1].wait()
        wb[ci] = pltpu.async_copy(
            ob, acc_sp.at[dbufs[ci % 3][0]], vbufs[ci % 3][1], add=True)
    wb[nch - 2].wait()
    wb[nch - 1].wait()
    plsc.subcore_barrier()
    pltpu.sync_copy(acc_sp.at[pl.ds(sid * NSL, NSL)],
                    out_hbm.at[cid, pl.ds(sid * NSL, NSL)])


def _glue1_body(w_ref, degp_ref, degsp_ref, x0_ref, x1_ref, dinv_ref, g1_ref):
    deg = (jnp.sum(degp_ref[...], axis=0) + jnp.sum(degsp_ref[...], axis=0)
           + 1.0)
    dinv = jax.lax.rsqrt(deg)
    z0 = x0_ref[...] * w_ref[0] + x1_ref[...] * w_ref[1]
    dinv_ref[...] = dinv
    g1_ref[...] = dinv * z0


def _glue2_body(c_ref, accp_ref, g1_ref, dinv_ref, g2_ref):
    s = jnp.sum(accp_ref[...], axis=0) + g1_ref[...]
    dinv = dinv_ref[...]
    z1 = dinv * s + c_ref[0]
    g2_ref[...] = dinv * z1


def _glue3_body(b_ref, accp_ref, g2_ref, dinv_ref, out_ref):
    s = jnp.sum(accp_ref[...], axis=0) + g2_ref[...]
    out_ref[...] = dinv_ref[...] * s + b_ref[0]


_part_spec = pl.BlockSpec((N_TILES, BK), lambda i: (0, i))
_part2_spec = pl.BlockSpec((2, BK), lambda i: (0, i))
_vec_spec = pl.BlockSpec((BK,), lambda i: (i,))
_smem_spec = pl.BlockSpec(memory_space=pltpu.SMEM)
_f1d = jax.ShapeDtypeStruct((N_PAD,), jnp.float32)

_glue1 = pl.pallas_call(
    _glue1_body, grid=(N_PAD // BK,),
    in_specs=[_smem_spec, _part_spec, _part2_spec, _vec_spec, _vec_spec],
    out_specs=[_vec_spec, _vec_spec],
    out_shape=[_f1d, _f1d],
)

_glue2 = pl.pallas_call(
    _glue2_body, grid=(N_PAD // BK,),
    in_specs=[_smem_spec, _part2_spec, _vec_spec, _vec_spec],
    out_specs=[_vec_spec],
    out_shape=[_f1d],
)

_glue3 = pl.pallas_call(
    _glue3_body, grid=(N_PAD // BK,),
    in_specs=[_smem_spec, _part2_spec, _vec_spec, _vec_spec],
    out_specs=[_vec_spec],
    out_shape=[_f1d],
)


def kernel(x, edge_index, W1, b1, W2, b2):
    src = edge_index[0].astype(jnp.int32)
    dst = edge_index[1].astype(jnp.int32)

    w = (W1 @ W2)[:, 0]
    c = jnp.dot(b1, W2[:, 0])

    xp = jnp.pad(x, ((0, N_PAD - N_NODES), (0, 0)))
    x0 = xp[:, 0]
    x1 = xp[:, 1]

    degp, degsp = _hist(dst)
    dinv, g1 = _glue1(w, degp, degsp, x0, x1)

    accp1 = _round(g1, src, dst)
    (g2,) = _glue2(jnp.reshape(c, (1,)), accp1, g1, dinv)

    accp2 = _round(g2, src, dst)
    (out,) = _glue3(b2, accp2, g2, dinv)

    return out[:N_NODES]
